# Initial kernel scaffold; baseline (speedup 1.0000x reference)
import jax
import jax.numpy as jnp
from jax.experimental import pallas as pl


def kernel(x1, edge_index1, e1, u1, batch1, x2, edge_index2, e2, u2, batch2, params):
    raise NotImplementedError("write your pallas kernel here")


# speedup vs baseline: 5.7407x; 5.7407x over previous
import functools

import jax
import jax.numpy as jnp
from jax import lax
from jax.experimental import pallas as pl
from jax.experimental.pallas import tpu as pltpu
from jax.experimental.pallas import tpu_sc as plsc

N_NODES = 10000
N_EDGES = 320000
N_GRAPHS = 8
BN = 1000
BE = 2000
CH = 128
NCHUNK = N_EDGES // CH
NW = 32
ROUNDS = (NCHUNK + NW - 1) // NW
SHN = 10240
NPS = SHN // 16
F32 = jnp.float32


def _onehot(b):
    return (b[:, None] == lax.broadcasted_iota(jnp.int32, (b.shape[0], N_GRAPHS), 1)).astype(F32)


def _dot(a, b):
    return jnp.dot(a, b, preferred_element_type=F32)


def _prep_body(u_ref, wu_ref, b1_ref, wa_ref, wb_ref, x_ref, batch_ref, ts_ref, td_ref):
    x = x_ref[...]
    uw = _dot(u_ref[...], wu_ref[...]) + b1_ref[...]
    oh = _onehot(batch_ref[0, 0, :])
    ts_ref[...] = _dot(x, wa_ref[...]) + _dot(oh, uw)
    td_ref[...] = _dot(x, wb_ref[...])


def _prep(x, batch3, u, wa, wb, wu, b1):
    return pl.pallas_call(
        _prep_body,
        grid=(N_NODES // BN,),
        in_specs=[
            pl.BlockSpec((N_GRAPHS, 128), lambda i: (0, 0)),
            pl.BlockSpec((128, 128), lambda i: (0, 0)),
            pl.BlockSpec((1, 128), lambda i: (0, 0)),
            pl.BlockSpec((128, 128), lambda i: (0, 0)),
            pl.BlockSpec((128, 128), lambda i: (0, 0)),
            pl.BlockSpec((BN, 128), lambda i: (i, 0)),
            pl.BlockSpec((1, 1, BN), lambda i: (i, 0, 0)),
        ],
        out_specs=[pl.BlockSpec((BN, 128), lambda i: (i, 0))] * 2,
        out_shape=[jax.ShapeDtypeStruct((N_NODES, 128), F32)] * 2,
    )(u, wu, b1, wa, wb, x, batch3)


def _edge_body(wc_ref, w2_ref, b2_ref, w3_ref, b3_ref, g_ref, e_ref, out_ref):
    h1 = jnp.maximum(g_ref[...] + _dot(e_ref[...][:, :16], wc_ref[...]), 0.0)
    h2 = jnp.maximum(_dot(h1, w2_ref[...]) + b2_ref[...], 0.0)
    o = _dot(h2, w3_ref[...]) + b3_ref[...]
    out_ref[...] = jnp.concatenate([o, jnp.ones((BE, 16), F32)], axis=1)


def _edge(g, e, wc, w2, b2, w3, b3):
    ecols = e.shape[1]
    return pl.pallas_call(
        _edge_body,
        grid=(N_EDGES // BE,),
        in_specs=[
            pl.BlockSpec((16, 128), lambda i: (0, 0)),
            pl.BlockSpec((128, 128), lambda i: (0, 0)),
            pl.BlockSpec((1, 128), lambda i: (0, 0)),
            pl.BlockSpec((128, 16), lambda i: (0, 0)),
            pl.BlockSpec((1, 16), lambda i: (0, 0)),
            pl.BlockSpec((BE, 128), lambda i: (i, 0)),
            pl.BlockSpec((BE, ecols), lambda i: (i, 0)),
        ],
        out_specs=pl.BlockSpec((BE, 32), lambda i: (i, 0)),
        out_shape=jax.ShapeDtypeStruct((N_EDGES, 32), F32),
    )(wc, w2, b2, w3, b3, g, e)


def _sc_mesh():
    return plsc.VectorSubcoreMesh(
        core_axis_name="c", subcore_axis_name="s", num_cores=2, num_subcores=16)


def _sc_gather(ts, td, src, dst):
    @functools.partial(
        pl.kernel,
        out_type=jax.ShapeDtypeStruct((N_EDGES, 128), F32),
        mesh=_sc_mesh(),
        scratch_types=[
            pltpu.VMEM((CH,), jnp.int32),
            pltpu.VMEM((CH,), jnp.int32),
            pltpu.VMEM((CH, 128), F32),
            pltpu.VMEM((CH, 128), F32),
            pltpu.SemaphoreType.DMA,
            pltpu.SemaphoreType.DMA,
        ],
    )
    def k(ts_hbm, td_hbm, src_hbm, dst_hbm, out_hbm, ia, ib, ba, bb, sa, sb):
        wid = lax.axis_index("s") * 2 + lax.axis_index("c")

        def round_(r, carry):
            chunk = r * NW + wid

            @pl.when(chunk < NCHUNK)
            def _():
                base = chunk * CH
                pltpu.sync_copy(src_hbm.at[pl.ds(base, CH)], ia)
                pltpu.sync_copy(dst_hbm.at[pl.ds(base, CH)], ib)
                ca = pltpu.async_copy(ts_hbm.at[ia], ba, sa)
                cb = pltpu.async_copy(td_hbm.at[ib], bb, sb)
                ca.wait()
                cb.wait()

                def addrow(i, c2):
                    for j in range(8):
                        sl = pl.ds(j * 16, 16)
                        ba[i, sl] = ba[i, sl] + bb[i, sl]
                    return c2

                lax.fori_loop(0, CH, addrow, 0)
                pltpu.sync_copy(ba, out_hbm.at[pl.ds(base, CH)])

            return carry

        lax.fori_loop(0, ROUNDS, round_, 0)

    return k(ts, td, src, dst)


def _sc_scatter(e32, dst):
    half = NCHUNK // 2

    @functools.partial(
        pl.kernel,
        out_type=jax.ShapeDtypeStruct((2 * SHN, 32), F32),
        mesh=_sc_mesh(),
        compiler_params=pltpu.CompilerParams(use_tc_tiling_on_sc=False),
        scratch_types=[
            pltpu.VMEM((CH,), jnp.int32),
            pltpu.VMEM((CH, 32), F32),
            pltpu.VMEM((NPS, 32), F32),
            pltpu.VMEM_SHARED((SHN, 32), F32),
        ],
    )
    def k(e_hbm, dst_hbm, out_hbm, idxb, rows, obuf, shared):
        cid = lax.axis_index("c")
        sid = lax.axis_index("s")

        def zr(i, c):
            obuf[i, pl.ds(0, 16)] = jnp.zeros((16,), F32)
            obuf[i, pl.ds(16, 16)] = jnp.zeros((16,), F32)
            return c

        lax.fori_loop(0, NPS, zr, 0)
        pltpu.sync_copy(obuf, shared.at[pl.ds(sid * NPS, NPS)])
        plsc.subcore_barrier()

        def round_(r, carry):
            cl = r * 16 + sid

            @pl.when(cl < half)
            def _():
                base = (cid * half + cl) * CH
                pltpu.sync_copy(dst_hbm.at[pl.ds(base, CH)], idxb)
                pltpu.sync_copy(e_hbm.at[pl.ds(base, CH)], rows)
                pltpu.sync_copy(rows, shared.at[idxb], add=True)

            return carry

        lax.fori_loop(0, (half + 15) // 16, round_, 0)
        plsc.subcore_barrier()
        pltpu.sync_copy(shared.at[pl.ds(sid * NPS, NPS)], obuf)
        pltpu.sync_copy(obuf, out_hbm.at[pl.ds(cid * SHN + sid * NPS, NPS)])

    return k(e32, dst)


def _node_body(u_ref, vu_ref, c1_ref, va_ref, vb_ref, v2_ref, c2_ref, v3_ref,
               c3_ref, x_ref, s0_ref, s1_ref, batch_ref, *rest, has_next):
    if has_next:
        wa_ref, wb_ref, xn_ref, xg_ref, cg_ref, tsp_ref, td_ref = rest
    else:
        xn_ref, xg_ref, cg_ref = rest
    s = s0_ref[...] + s1_ref[...]
    agg = s[:, :16] / jnp.maximum(s[:, 16:17], 1.0)
    oh = _onehot(batch_ref[0, 0, :])
    uw = _dot(u_ref[...], vu_ref[...]) + c1_ref[...]
    a1 = jnp.maximum(_dot(x_ref[...], va_ref[...]) + _dot(agg, vb_ref[...]) + _dot(oh, uw), 0.0)
    a2 = jnp.maximum(_dot(a1, v2_ref[...]) + c2_ref[...], 0.0)
    xn = _dot(a2, v3_ref[...]) + c3_ref[...]
    xn_ref[...] = xn
    pg = lax.dot_general(oh, xn, (((0,), (0,)), ((), ())), preferred_element_type=F32)
    cg = lax.dot_general(oh, jnp.ones_like(xn), (((0,), (0,)), ((), ())),
                         preferred_element_type=F32)

    @pl.when(pl.program_id(0) == 0)
    def _():
        xg_ref[...] = pg
        cg_ref[...] = cg

    @pl.when(pl.program_id(0) != 0)
    def _():
        xg_ref[...] = xg_ref[...] + pg
        cg_ref[...] = cg_ref[...] + cg

    if has_next:
        tsp_ref[...] = _dot(xn, wa_ref[...])
        td_ref[...] = _dot(xn, wb_ref[...])


def _node(x, sc, u, batch3, w, enext, has_next):
    cfull = lambda shape: pl.BlockSpec(shape, lambda i: tuple(0 for _ in shape))
    nblk = pl.BlockSpec((BN, 128), lambda i: (i, 0))
    accb = pl.BlockSpec((N_GRAPHS, 128), lambda i: (0, 0))
    in_specs = [
        cfull((N_GRAPHS, 128)), cfull((128, 128)), cfull((1, 128)),
        cfull((128, 128)), cfull((16, 128)), cfull((128, 128)), cfull((1, 128)),
        cfull((128, 128)), cfull((1, 128)),
        nblk,
        pl.BlockSpec((BN, 32), lambda i: (i, 0)),
        pl.BlockSpec((BN, 32), lambda i: (i, 0)),
        pl.BlockSpec((1, 1, BN), lambda i: (i, 0, 0)),
    ]
    args = [u, w['nvu'], w['nc1'], w['nva'], w['nvb'], w['nv2'], w['nc2'],
            w['nv3'], w['nc3'], x, sc[:N_NODES], sc[SHN:SHN + N_NODES], batch3]
    out_specs = [nblk, accb, accb]
    out_shape = [jax.ShapeDtypeStruct((N_NODES, 128), F32),
                 jax.ShapeDtypeStruct((N_GRAPHS, 128), F32),
                 jax.ShapeDtypeStruct((N_GRAPHS, 128), F32)]
    if has_next:
        in_specs += [cfull((128, 128)), cfull((128, 128))]
        args += [enext['ewa'], enext['ewb']]
        out_specs += [nblk, nblk]
        out_shape += [jax.ShapeDtypeStruct((N_NODES, 128), F32)] * 2
    return pl.pallas_call(
        functools.partial(_node_body, has_next=has_next),
        grid=(N_NODES // BN,),
        in_specs=in_specs, out_specs=out_specs, out_shape=out_shape,
    )(*args)


def _glob_body(xg1_ref, cg1_ref, u1_ref, xg2_ref, cg2_ref, u2_ref,
               ga1, gb1, h11, g21, h21, g31, h31,
               ga2, gb2, h12, g22, h22, g32, h32,
               ma, mb, n1, m2, n2, m3, n3, *rest, has_next):
    if has_next:
        wu1, eb1, wu2, eb2, u1n_ref, u2n_ref, out_ref, tsu1_ref, tsu2_ref = rest
    else:
        u1n_ref, u2n_ref, out_ref = rest

    def gmlp(xg_ref, cg_ref, u_ref, ga, gb, hb1, g2, hb2, g3, hb3):
        xg = xg_ref[...] / jnp.maximum(cg_ref[...], 1.0)
        t1 = jnp.maximum(_dot(xg, ga[...]) + _dot(u_ref[...], gb[...]) + hb1[...], 0.0)
        t2 = jnp.maximum(_dot(t1, g2[...]) + hb2[...], 0.0)
        return _dot(t2, g3[...]) + hb3[...]

    u1n = gmlp(xg1_ref, cg1_ref, u1_ref, ga1, gb1, h11, g21, h21, g31, h31)
    u2n = gmlp(xg2_ref, cg2_ref, u2_ref, ga2, gb2, h12, g22, h22, g32, h32)
    u1n_ref[...] = u1n
    u2n_ref[...] = u2n
    m1v = jnp.maximum(_dot(u1n, ma[...]) + _dot(u2n, mb[...]) + n1[...], 0.0)
    m2v = jnp.maximum(_dot(m1v, m2[...]) + n2[...], 0.0)
    out_ref[...] = _dot(m2v, m3[...]) + n3[...]
    if has_next:
        tsu1_ref[...] = _dot(u1n, wu1[...]) + eb1[...]
        tsu2_ref[...] = _dot(u2n, wu2[...]) + eb2[...]


def _glob(xg1, cg1, u1, xg2, cg2, u2, w1, w2, mw, e1w, e2w, has_next):
    args = [xg1, cg1, u1, xg2, cg2, u2,
            w1['gga'], w1['ggb'], w1['gd1'], w1['gg2'], w1['gd2'], w1['gg3'], w1['gd3'],
            w2['gga'], w2['ggb'], w2['gd1'], w2['gg2'], w2['gd2'], w2['gg3'], w2['gd3'],
            mw['ma'], mw['mb'], mw['n1'], mw['m2'], mw['n2'], mw['m3p'], mw['n3p']]
    nout = 3
    out_shape = [jax.ShapeDtypeStruct((N_GRAPHS, 128), F32)] * 3
    if has_next:
        args += [e1w['ewu'], e1w['eb1'], e2w['ewu'], e2w['eb1']]
        out_shape += [jax.ShapeDtypeStruct((N_GRAPHS, 128), F32)] * 2
        nout = 5
    return pl.pallas_call(
        functools.partial(_glob_body, has_next=has_next),
        out_shape=out_shape,
    )(*args)


def _tfix_body(tsu1_ref, tsu2_ref, tsp1_ref, b1_ref, tsp2_ref, b2_ref,
               ts1_ref, ts2_ref):
    oh1 = _onehot(b1_ref[0, 0, :])
    oh2 = _onehot(b2_ref[0, 0, :])
    ts1_ref[...] = tsp1_ref[...] + _dot(oh1, tsu1_ref[...])
    ts2_ref[...] = tsp2_ref[...] + _dot(oh2, tsu2_ref[...])


def _tfix(tsp1, batch31, tsu1, tsp2, batch32, tsu2):
    cfull = lambda shape: pl.BlockSpec(shape, lambda i: tuple(0 for _ in shape))
    nblk = pl.BlockSpec((BN, 128), lambda i: (i, 0))
    bblk = pl.BlockSpec((1, 1, BN), lambda i: (i, 0, 0))
    return pl.pallas_call(
        _tfix_body,
        grid=(N_NODES // BN,),
        in_specs=[cfull((N_GRAPHS, 128)), cfull((N_GRAPHS, 128)),
                  nblk, bblk, nblk, bblk],
        out_specs=[nblk, nblk],
        out_shape=[jax.ShapeDtypeStruct((N_NODES, 128), F32)] * 2,
    )(tsu1, tsu2, tsp1, batch31, tsp2, batch32)


def _unpack_block(p):
    (W1, b1), (W2, b2), (W3, b3) = p['edge']
    (V1, c1), (V2, c2), (V3, c3) = p['node']
    (G1, d1), (G2, d2), (G3, d3) = p['global']
    return dict(
        ewa=W1[0:128], ewb=W1[128:256], ewc=W1[256:272], ewu=W1[272:400],
        eb1=b1.reshape(1, 128), ew2=W2, eb2=b2.reshape(1, 128),
        ew3=W3, eb3=b3.reshape(1, 16),
        nva=V1[0:128], nvb=V1[128:144], nvu=V1[144:272], nc1=c1.reshape(1, 128),
        nv2=V2, nc2=c2.reshape(1, 128), nv3=V3, nc3=c3.reshape(1, 128),
        gga=G1[0:128], ggb=G1[128:256], gd1=d1.reshape(1, 128),
        gg2=G2, gd2=d2.reshape(1, 128), gg3=G3, gd3=d3.reshape(1, 128),
    )


def _unpack_mlp(pm):
    (M1, n1), (M2, n2), (M3, n3) = pm
    m3p = jnp.zeros((128, 128), F32).at[:, :2].set(M3)
    n3p = jnp.zeros((1, 128), F32).at[0, :2].set(n3)
    return dict(ma=M1[:128], mb=M1[128:], n1=n1.reshape(1, 128),
                m2=M2, n2=n2.reshape(1, 128), m3p=m3p, n3p=n3p)


def kernel(x1, edge_index1, e1, u1, batch1, x2, edge_index2, e2, u2, batch2, params):
    w1 = _unpack_block(params['gnn1'])
    w2 = _unpack_block(params['gnn2'])
    mw = _unpack_mlp(params['mlp'])
    s1, d1 = edge_index1[0], edge_index1[1]
    s2, d2 = edge_index2[0], edge_index2[1]
    b31 = batch1.reshape(N_NODES // BN, 1, BN)
    b32 = batch2.reshape(N_NODES // BN, 1, BN)

    ts1, td1 = _prep(x1, b31, u1, w1['ewa'], w1['ewb'], w1['ewu'], w1['eb1'])
    ts2, td2 = _prep(x2, b32, u2, w2['ewa'], w2['ewb'], w2['ewu'], w2['eb1'])

    outs = []
    e1c, e2c = e1, e2
    x1c, x2c = x1, x2
    u1c, u2c = u1, u2
    for step in range(2):
        has_next = step == 0
        g1 = _sc_gather(ts1, td1, s1, d1)
        g2 = _sc_gather(ts2, td2, s2, d2)
        en1 = _edge(g1, e1c, w1['ewc'], w1['ew2'], w1['eb2'], w1['ew3'], w1['eb3'])
        en2 = _edge(g2, e2c, w2['ewc'], w2['ew2'], w2['eb2'], w2['ew3'], w2['eb3'])
        sc1 = _sc_scatter(en1, d1)
        sc2 = _sc_scatter(en2, d2)
        r1 = _node(x1c, sc1, u1c, b31, w1, w1, has_next)
        r2 = _node(x2c, sc2, u2c, b32, w2, w2, has_next)
        gout = _glob(r1[1], r1[2], u1c, r2[1], r2[2], u2c, w1, w2, mw, w1, w2, has_next)
        outs.append(gout[2][:, :2])
        if has_next:
            ts1, ts2 = _tfix(r1[3], b31, gout[3], r2[3], b32, gout[4])
            td1, td2 = r1[4], r2[4]
        x1c, x2c = r1[0], r2[0]
        u1c, u2c = gout[0], gout[1]
        e1c, e2c = en1, en2
    return jnp.stack(outs)


# speedup vs baseline: 6.4558x; 1.1246x over previous
import functools

import jax
import jax.numpy as jnp
from jax import lax
from jax.experimental import pallas as pl
from jax.experimental.pallas import tpu as pltpu
from jax.experimental.pallas import tpu_sc as plsc

N_NODES = 10000
N_EDGES = 320000
N_GRAPHS = 8
BN = 1000
BE = 2000
CH = 128
NCHUNK = N_EDGES // CH
NW = 32
ROUNDS = (NCHUNK + NW - 1) // NW
SHN = 10240
NPS = SHN // 16
F32 = jnp.float32


def _onehot(b):
    return (b[:, None] == lax.broadcasted_iota(jnp.int32, (b.shape[0], N_GRAPHS), 1)).astype(F32)


def _dot(a, b):
    return jnp.dot(a, b, preferred_element_type=F32)


def _prep_body(u_ref, wu_ref, b1_ref, wa_ref, wb_ref, x_ref, batch_ref, ts_ref, td_ref):
    x = x_ref[...]
    uw = _dot(u_ref[...], wu_ref[...]) + b1_ref[...]
    oh = _onehot(batch_ref[0, 0, :])
    ts_ref[...] = _dot(x, wa_ref[...]) + _dot(oh, uw)
    td_ref[...] = _dot(x, wb_ref[...])


def _prep(x, batch3, u, wa, wb, wu, b1):
    return pl.pallas_call(
        _prep_body,
        grid=(N_NODES // BN,),
        in_specs=[
            pl.BlockSpec((N_GRAPHS, 128), lambda i: (0, 0)),
            pl.BlockSpec((128, 128), lambda i: (0, 0)),
            pl.BlockSpec((1, 128), lambda i: (0, 0)),
            pl.BlockSpec((128, 128), lambda i: (0, 0)),
            pl.BlockSpec((128, 128), lambda i: (0, 0)),
            pl.BlockSpec((BN, 128), lambda i: (i, 0)),
            pl.BlockSpec((1, 1, BN), lambda i: (i, 0, 0)),
        ],
        out_specs=[pl.BlockSpec((BN, 128), lambda i: (i, 0))] * 2,
        out_shape=[jax.ShapeDtypeStruct((N_NODES, 128), F32)] * 2,
    )(u, wu, b1, wa, wb, x, batch3)


def _edge_body(wc_ref, w2_ref, b2_ref, w3_ref, b3_ref, g_ref, e_ref, out_ref):
    h1 = jnp.maximum(g_ref[...] + _dot(e_ref[...][:, :16], wc_ref[...]), 0.0)
    h2 = jnp.maximum(_dot(h1, w2_ref[...]) + b2_ref[...], 0.0)
    o = _dot(h2, w3_ref[...]) + b3_ref[...]
    out_ref[...] = jnp.concatenate([o, jnp.ones((BE, 16), F32)], axis=1)


def _edge(g, e, wc, w2, b2, w3, b3):
    ecols = e.shape[1]
    return pl.pallas_call(
        _edge_body,
        grid=(N_EDGES // BE,),
        in_specs=[
            pl.BlockSpec((16, 128), lambda i: (0, 0)),
            pl.BlockSpec((128, 128), lambda i: (0, 0)),
            pl.BlockSpec((1, 128), lambda i: (0, 0)),
            pl.BlockSpec((128, 16), lambda i: (0, 0)),
            pl.BlockSpec((1, 16), lambda i: (0, 0)),
            pl.BlockSpec((BE, 128), lambda i: (i, 0)),
            pl.BlockSpec((BE, ecols), lambda i: (i, 0)),
        ],
        out_specs=pl.BlockSpec((BE, 32), lambda i: (i, 0)),
        out_shape=jax.ShapeDtypeStruct((N_EDGES, 32), F32),
    )(wc, w2, b2, w3, b3, g, e)


def _sc_mesh():
    return plsc.VectorSubcoreMesh(
        core_axis_name="c", subcore_axis_name="s", num_cores=2, num_subcores=16)


def _sc_gather(ts, td, src, dst):
    @functools.partial(
        pl.kernel,
        out_type=jax.ShapeDtypeStruct((N_EDGES, 128), F32),
        mesh=_sc_mesh(),
        scratch_types=[
            pltpu.VMEM((2, CH), jnp.int32),
            pltpu.VMEM((2, CH), jnp.int32),
            pltpu.VMEM((CH, 128), F32),
            pltpu.VMEM((CH, 128), F32),
            pltpu.VMEM((CH, 128), F32),
            pltpu.VMEM((CH, 128), F32),
            pltpu.SemaphoreType.DMA,
            pltpu.SemaphoreType.DMA,
            pltpu.SemaphoreType.DMA,
            pltpu.SemaphoreType.DMA,
            pltpu.SemaphoreType.DMA,
            pltpu.SemaphoreType.DMA,
        ],
    )
    def k(ts_hbm, td_hbm, src_hbm, dst_hbm, out_hbm,
          ia, ib, a0, b0, a1, b1, ga0, ga1, gb0, gb1, ix0, ix1):
        wid = lax.axis_index("s") * 2 + lax.axis_index("c")
        ab = (a0, a1)
        bb_ = (b0, b1)
        ga = (ga0, ga1)
        gb = (gb0, gb1)
        ix = (ix0, ix1)

        def issue(t, p):
            chunk = t * NW + wid

            @pl.when((t < ROUNDS) & (chunk < NCHUNK))
            def _():
                base = chunk * CH

                @pl.when(t == 0)
                def _():
                    pltpu.sync_copy(src_hbm.at[pl.ds(base, CH)], ia.at[p])
                    pltpu.sync_copy(dst_hbm.at[pl.ds(base, CH)], ib.at[p])

                @pl.when(t > 0)
                def _():
                    pltpu.make_async_copy(src_hbm.at[pl.ds(base, CH)], ia.at[p], ix[p]).wait()
                    pltpu.make_async_copy(dst_hbm.at[pl.ds(base, CH)], ib.at[p], ix[p]).wait()

                pltpu.async_copy(ts_hbm.at[ia.at[p]], ab[p], ga[p])
                pltpu.async_copy(td_hbm.at[ib.at[p]], bb_[p], gb[p])

        def complete(t, p):
            chunk = t * NW + wid

            @pl.when((t >= 0) & (chunk < NCHUNK))
            def _():
                base = chunk * CH
                pltpu.make_async_copy(ts_hbm.at[pl.ds(0, CH)], ab[p], ga[p]).wait()
                pltpu.make_async_copy(td_hbm.at[pl.ds(0, CH)], bb_[p], gb[p]).wait()
                a_ref = ab[p]
                b_ref = bb_[p]

                def addrow(i, c2):
                    for j in range(8):
                        sl = pl.ds(j * 16, 16)
                        a_ref[i, sl] = a_ref[i, sl] + b_ref[i, sl]
                    return c2

                lax.fori_loop(0, CH, addrow, 0)
                pltpu.sync_copy(a_ref, out_hbm.at[pl.ds(base, CH)])

        def prefetch(t, p):
            chunk = t * NW + wid

            @pl.when((t < ROUNDS) & (chunk < NCHUNK))
            def _():
                base = chunk * CH
                pltpu.async_copy(src_hbm.at[pl.ds(base, CH)], ia.at[p], ix[p])
                pltpu.async_copy(dst_hbm.at[pl.ds(base, CH)], ib.at[p], ix[p])

        def round_(t, carry):
            for p in (0, 1):
                @pl.when((t % 2) == p)
                def _(p=p):
                    issue(t, p)
                    complete(t - 1, 1 - p)
                    prefetch(t + 1, 1 - p)
            return carry

        lax.fori_loop(0, ROUNDS + 1, round_, 0)

    return k(ts, td, src, dst)


def _sc_scatter(e32, dst):
    half = NCHUNK // 2

    @functools.partial(
        pl.kernel,
        out_type=jax.ShapeDtypeStruct((2 * SHN, 32), F32),
        mesh=_sc_mesh(),
        compiler_params=pltpu.CompilerParams(use_tc_tiling_on_sc=False),
        scratch_types=[
            pltpu.VMEM((CH,), jnp.int32),
            pltpu.VMEM((CH, 32), F32),
            pltpu.VMEM((NPS, 32), F32),
            pltpu.VMEM_SHARED((SHN, 32), F32),
        ],
    )
    def k(e_hbm, dst_hbm, out_hbm, idxb, rows, obuf, shared):
        cid = lax.axis_index("c")
        sid = lax.axis_index("s")

        def zr(i, c):
            obuf[i, pl.ds(0, 16)] = jnp.zeros((16,), F32)
            obuf[i, pl.ds(16, 16)] = jnp.zeros((16,), F32)
            return c

        lax.fori_loop(0, NPS, zr, 0)
        pltpu.sync_copy(obuf, shared.at[pl.ds(sid * NPS, NPS)])
        plsc.subcore_barrier()

        def round_(r, carry):
            cl = r * 16 + sid

            @pl.when(cl < half)
            def _():
                base = (cid * half + cl) * CH
                pltpu.sync_copy(dst_hbm.at[pl.ds(base, CH)], idxb)
                pltpu.sync_copy(e_hbm.at[pl.ds(base, CH)], rows)
                pltpu.sync_copy(rows, shared.at[idxb], add=True)

            return carry

        lax.fori_loop(0, (half + 15) // 16, round_, 0)
        plsc.subcore_barrier()
        pltpu.sync_copy(shared.at[pl.ds(sid * NPS, NPS)], obuf)
        pltpu.sync_copy(obuf, out_hbm.at[pl.ds(cid * SHN + sid * NPS, NPS)])

    return k(e32, dst)


def _node_body(u_ref, vu_ref, c1_ref, va_ref, vb_ref, v2_ref, c2_ref, v3_ref,
               c3_ref, x_ref, s0_ref, s1_ref, batch_ref, *rest, has_next):
    if has_next:
        wa_ref, wb_ref, xn_ref, xg_ref, cg_ref, tsp_ref, td_ref = rest
    else:
        xn_ref, xg_ref, cg_ref = rest
    s = s0_ref[...] + s1_ref[...]
    agg = s[:, :16] / jnp.maximum(s[:, 16:17], 1.0)
    oh = _onehot(batch_ref[0, 0, :])
    uw = _dot(u_ref[...], vu_ref[...]) + c1_ref[...]
    a1 = jnp.maximum(_dot(x_ref[...], va_ref[...]) + _dot(agg, vb_ref[...]) + _dot(oh, uw), 0.0)
    a2 = jnp.maximum(_dot(a1, v2_ref[...]) + c2_ref[...], 0.0)
    xn = _dot(a2, v3_ref[...]) + c3_ref[...]
    xn_ref[...] = xn
    pg = lax.dot_general(oh, xn, (((0,), (0,)), ((), ())), preferred_element_type=F32)
    cg = lax.dot_general(oh, jnp.ones_like(xn), (((0,), (0,)), ((), ())),
                         preferred_element_type=F32)

    @pl.when(pl.program_id(0) == 0)
    def _():
        xg_ref[...] = pg
        cg_ref[...] = cg

    @pl.when(pl.program_id(0) != 0)
    def _():
        xg_ref[...] = xg_ref[...] + pg
        cg_ref[...] = cg_ref[...] + cg

    if has_next:
        tsp_ref[...] = _dot(xn, wa_ref[...])
        td_ref[...] = _dot(xn, wb_ref[...])


def _node(x, sc, u, batch3, w, enext, has_next):
    cfull = lambda shape: pl.BlockSpec(shape, lambda i: tuple(0 for _ in shape))
    nblk = pl.BlockSpec((BN, 128), lambda i: (i, 0))
    accb = pl.BlockSpec((N_GRAPHS, 128), lambda i: (0, 0))
    in_specs = [
        cfull((N_GRAPHS, 128)), cfull((128, 128)), cfull((1, 128)),
        cfull((128, 128)), cfull((16, 128)), cfull((128, 128)), cfull((1, 128)),
        cfull((128, 128)), cfull((1, 128)),
        nblk,
        pl.BlockSpec((BN, 32), lambda i: (i, 0)),
        pl.BlockSpec((BN, 32), lambda i: (i, 0)),
        pl.BlockSpec((1, 1, BN), lambda i: (i, 0, 0)),
    ]
    args = [u, w['nvu'], w['nc1'], w['nva'], w['nvb'], w['nv2'], w['nc2'],
            w['nv3'], w['nc3'], x, sc[:N_NODES], sc[SHN:SHN + N_NODES], batch3]
    out_specs = [nblk, accb, accb]
    out_shape = [jax.ShapeDtypeStruct((N_NODES, 128), F32),
                 jax.ShapeDtypeStruct((N_GRAPHS, 128), F32),
                 jax.ShapeDtypeStruct((N_GRAPHS, 128), F32)]
    if has_next:
        in_specs += [cfull((128, 128)), cfull((128, 128))]
        args += [enext['ewa'], enext['ewb']]
        out_specs += [nblk, nblk]
        out_shape += [jax.ShapeDtypeStruct((N_NODES, 128), F32)] * 2
    return pl.pallas_call(
        functools.partial(_node_body, has_next=has_next),
        grid=(N_NODES // BN,),
        in_specs=in_specs, out_specs=out_specs, out_shape=out_shape,
    )(*args)


def _glob_body(xg1_ref, cg1_ref, u1_ref, xg2_ref, cg2_ref, u2_ref,
               ga1, gb1, h11, g21, h21, g31, h31,
               ga2, gb2, h12, g22, h22, g32, h32,
               ma, mb, n1, m2, n2, m3, n3, *rest, has_next):
    if has_next:
        wu1, eb1, wu2, eb2, u1n_ref, u2n_ref, out_ref, tsu1_ref, tsu2_ref = rest
    else:
        u1n_ref, u2n_ref, out_ref = rest

    def gmlp(xg_ref, cg_ref, u_ref, ga, gb, hb1, g2, hb2, g3, hb3):
        xg = xg_ref[...] / jnp.maximum(cg_ref[...], 1.0)
        t1 = jnp.maximum(_dot(xg, ga[...]) + _dot(u_ref[...], gb[...]) + hb1[...], 0.0)
        t2 = jnp.maximum(_dot(t1, g2[...]) + hb2[...], 0.0)
        return _dot(t2, g3[...]) + hb3[...]

    u1n = gmlp(xg1_ref, cg1_ref, u1_ref, ga1, gb1, h11, g21, h21, g31, h31)
    u2n = gmlp(xg2_ref, cg2_ref, u2_ref, ga2, gb2, h12, g22, h22, g32, h32)
    u1n_ref[...] = u1n
    u2n_ref[...] = u2n
    m1v = jnp.maximum(_dot(u1n, ma[...]) + _dot(u2n, mb[...]) + n1[...], 0.0)
    m2v = jnp.maximum(_dot(m1v, m2[...]) + n2[...], 0.0)
    out_ref[...] = _dot(m2v, m3[...]) + n3[...]
    if has_next:
        tsu1_ref[...] = _dot(u1n, wu1[...]) + eb1[...]
        tsu2_ref[...] = _dot(u2n, wu2[...]) + eb2[...]


def _glob(xg1, cg1, u1, xg2, cg2, u2, w1, w2, mw, e1w, e2w, has_next):
    args = [xg1, cg1, u1, xg2, cg2, u2,
            w1['gga'], w1['ggb'], w1['gd1'], w1['gg2'], w1['gd2'], w1['gg3'], w1['gd3'],
            w2['gga'], w2['ggb'], w2['gd1'], w2['gg2'], w2['gd2'], w2['gg3'], w2['gd3'],
            mw['ma'], mw['mb'], mw['n1'], mw['m2'], mw['n2'], mw['m3p'], mw['n3p']]
    nout = 3
    out_shape = [jax.ShapeDtypeStruct((N_GRAPHS, 128), F32)] * 3
    if has_next:
        args += [e1w['ewu'], e1w['eb1'], e2w['ewu'], e2w['eb1']]
        out_shape += [jax.ShapeDtypeStruct((N_GRAPHS, 128), F32)] * 2
        nout = 5
    return pl.pallas_call(
        functools.partial(_glob_body, has_next=has_next),
        out_shape=out_shape,
    )(*args)


def _tfix_body(tsu1_ref, tsu2_ref, tsp1_ref, b1_ref, tsp2_ref, b2_ref,
               ts1_ref, ts2_ref):
    oh1 = _onehot(b1_ref[0, 0, :])
    oh2 = _onehot(b2_ref[0, 0, :])
    ts1_ref[...] = tsp1_ref[...] + _dot(oh1, tsu1_ref[...])
    ts2_ref[...] = tsp2_ref[...] + _dot(oh2, tsu2_ref[...])


def _tfix(tsp1, batch31, tsu1, tsp2, batch32, tsu2):
    cfull = lambda shape: pl.BlockSpec(shape, lambda i: tuple(0 for _ in shape))
    nblk = pl.BlockSpec((BN, 128), lambda i: (i, 0))
    bblk = pl.BlockSpec((1, 1, BN), lambda i: (i, 0, 0))
    return pl.pallas_call(
        _tfix_body,
        grid=(N_NODES // BN,),
        in_specs=[cfull((N_GRAPHS, 128)), cfull((N_GRAPHS, 128)),
                  nblk, bblk, nblk, bblk],
        out_specs=[nblk, nblk],
        out_shape=[jax.ShapeDtypeStruct((N_NODES, 128), F32)] * 2,
    )(tsu1, tsu2, tsp1, batch31, tsp2, batch32)


def _unpack_block(p):
    (W1, b1), (W2, b2), (W3, b3) = p['edge']
    (V1, c1), (V2, c2), (V3, c3) = p['node']
    (G1, d1), (G2, d2), (G3, d3) = p['global']
    return dict(
        ewa=W1[0:128], ewb=W1[128:256], ewc=W1[256:272], ewu=W1[272:400],
        eb1=b1.reshape(1, 128), ew2=W2, eb2=b2.reshape(1, 128),
        ew3=W3, eb3=b3.reshape(1, 16),
        nva=V1[0:128], nvb=V1[128:144], nvu=V1[144:272], nc1=c1.reshape(1, 128),
        nv2=V2, nc2=c2.reshape(1, 128), nv3=V3, nc3=c3.reshape(1, 128),
        gga=G1[0:128], ggb=G1[128:256], gd1=d1.reshape(1, 128),
        gg2=G2, gd2=d2.reshape(1, 128), gg3=G3, gd3=d3.reshape(1, 128),
    )


def _unpack_mlp(pm):
    (M1, n1), (M2, n2), (M3, n3) = pm
    m3p = jnp.zeros((128, 128), F32).at[:, :2].set(M3)
    n3p = jnp.zeros((1, 128), F32).at[0, :2].set(n3)
    return dict(ma=M1[:128], mb=M1[128:], n1=n1.reshape(1, 128),
                m2=M2, n2=n2.reshape(1, 128), m3p=m3p, n3p=n3p)


def kernel(x1, edge_index1, e1, u1, batch1, x2, edge_index2, e2, u2, batch2, params):
    w1 = _unpack_block(params['gnn1'])
    w2 = _unpack_block(params['gnn2'])
    mw = _unpack_mlp(params['mlp'])
    s1, d1 = edge_index1[0], edge_index1[1]
    s2, d2 = edge_index2[0], edge_index2[1]
    b31 = batch1.reshape(N_NODES // BN, 1, BN)
    b32 = batch2.reshape(N_NODES // BN, 1, BN)

    ts1, td1 = _prep(x1, b31, u1, w1['ewa'], w1['ewb'], w1['ewu'], w1['eb1'])
    ts2, td2 = _prep(x2, b32, u2, w2['ewa'], w2['ewb'], w2['ewu'], w2['eb1'])

    outs = []
    e1c, e2c = e1, e2
    x1c, x2c = x1, x2
    u1c, u2c = u1, u2
    for step in range(2):
        has_next = step == 0
        g1 = _sc_gather(ts1, td1, s1, d1)
        g2 = _sc_gather(ts2, td2, s2, d2)
        en1 = _edge(g1, e1c, w1['ewc'], w1['ew2'], w1['eb2'], w1['ew3'], w1['eb3'])
        en2 = _edge(g2, e2c, w2['ewc'], w2['ew2'], w2['eb2'], w2['ew3'], w2['eb3'])
        sc1 = _sc_scatter(en1, d1)
        sc2 = _sc_scatter(en2, d2)
        r1 = _node(x1c, sc1, u1c, b31, w1, w1, has_next)
        r2 = _node(x2c, sc2, u2c, b32, w2, w2, has_next)
        gout = _glob(r1[1], r1[2], u1c, r2[1], r2[2], u2c, w1, w2, mw, w1, w2, has_next)
        outs.append(gout[2][:, :2])
        if has_next:
            ts1, ts2 = _tfix(r1[3], b31, gout[3], r2[3], b32, gout[4])
            td1, td2 = r1[4], r2[4]
        x1c, x2c = r1[0], r2[0]
        u1c, u2c = gout[0], gout[1]
        e1c, e2c = en1, en2
    return jnp.stack(outs)


# speedup vs baseline: 6.7245x; 1.0416x over previous
import functools

import jax
import jax.numpy as jnp
from jax import lax
from jax.experimental import pallas as pl
from jax.experimental.pallas import tpu as pltpu
from jax.experimental.pallas import tpu_sc as plsc

N_NODES = 10000
N_EDGES = 320000
N_GRAPHS = 8
BN = 1000
BE = 2000
CH = 128
NCHUNK = N_EDGES // CH
NW = 32
ROUNDS = (NCHUNK + NW - 1) // NW
SHN = 10240
NPS = SHN // 16
F32 = jnp.float32


def _onehot(b):
    return (b[:, None] == lax.broadcasted_iota(jnp.int32, (b.shape[0], N_GRAPHS), 1)).astype(F32)


def _dot(a, b):
    return jnp.dot(a, b, preferred_element_type=F32)


def _prep_body(u_ref, wu_ref, b1_ref, wa_ref, wb_ref, x_ref, batch_ref, ts_ref, td_ref):
    x = x_ref[...]
    uw = _dot(u_ref[...], wu_ref[...]) + b1_ref[...]
    oh = _onehot(batch_ref[0, 0, :])
    ts_ref[...] = _dot(x, wa_ref[...]) + _dot(oh, uw)
    td_ref[...] = _dot(x, wb_ref[...])


def _prep(x, batch3, u, wa, wb, wu, b1):
    return pl.pallas_call(
        _prep_body,
        grid=(N_NODES // BN,),
        in_specs=[
            pl.BlockSpec((N_GRAPHS, 128), lambda i: (0, 0)),
            pl.BlockSpec((128, 128), lambda i: (0, 0)),
            pl.BlockSpec((1, 128), lambda i: (0, 0)),
            pl.BlockSpec((128, 128), lambda i: (0, 0)),
            pl.BlockSpec((128, 128), lambda i: (0, 0)),
            pl.BlockSpec((BN, 128), lambda i: (i, 0)),
            pl.BlockSpec((1, 1, BN), lambda i: (i, 0, 0)),
        ],
        out_specs=[pl.BlockSpec((BN, 128), lambda i: (i, 0))] * 2,
        out_shape=[jax.ShapeDtypeStruct((N_NODES, 128), F32)] * 2,
    )(u, wu, b1, wa, wb, x, batch3)


def _edge_body(wc_ref, w2_ref, b2_ref, w3_ref, b3_ref, g_ref, e_ref, out_ref):
    h1 = jnp.maximum(g_ref[...] + _dot(e_ref[...][:, :16], wc_ref[...]), 0.0)
    h2 = jnp.maximum(_dot(h1, w2_ref[...]) + b2_ref[...], 0.0)
    o = _dot(h2, w3_ref[...]) + b3_ref[...]
    out_ref[...] = jnp.concatenate([o, jnp.ones((BE, 16), F32)], axis=1)


def _edge(g, e, wc, w2, b2, w3, b3):
    ecols = e.shape[1]
    return pl.pallas_call(
        _edge_body,
        grid=(N_EDGES // BE,),
        in_specs=[
            pl.BlockSpec((16, 128), lambda i: (0, 0)),
            pl.BlockSpec((128, 128), lambda i: (0, 0)),
            pl.BlockSpec((1, 128), lambda i: (0, 0)),
            pl.BlockSpec((128, 16), lambda i: (0, 0)),
            pl.BlockSpec((1, 16), lambda i: (0, 0)),
            pl.BlockSpec((BE, 128), lambda i: (i, 0)),
            pl.BlockSpec((BE, ecols), lambda i: (i, 0)),
        ],
        out_specs=pl.BlockSpec((BE, 32), lambda i: (i, 0)),
        out_shape=jax.ShapeDtypeStruct((N_EDGES, 32), F32),
    )(wc, w2, b2, w3, b3, g, e)


def _sc_mesh():
    return plsc.VectorSubcoreMesh(
        core_axis_name="c", subcore_axis_name="s", num_cores=2, num_subcores=16)


def _sc_gather(ts, td, src, dst):
    @functools.partial(
        pl.kernel,
        out_type=jax.ShapeDtypeStruct((N_EDGES, 128), F32),
        mesh=_sc_mesh(),
        scratch_types=[
            pltpu.VMEM((2, CH), jnp.int32),
            pltpu.VMEM((2, CH), jnp.int32),
            pltpu.VMEM((CH, 128), F32),
            pltpu.VMEM((CH, 128), F32),
            pltpu.VMEM((CH, 128), F32),
            pltpu.VMEM((CH, 128), F32),
            pltpu.SemaphoreType.DMA,
            pltpu.SemaphoreType.DMA,
            pltpu.SemaphoreType.DMA,
            pltpu.SemaphoreType.DMA,
            pltpu.SemaphoreType.DMA,
            pltpu.SemaphoreType.DMA,
        ],
    )
    def k(ts_hbm, td_hbm, src_hbm, dst_hbm, out_hbm,
          ia, ib, a0, b0, a1, b1, ga0, ga1, gb0, gb1, ix0, ix1):
        wid = lax.axis_index("s") * 2 + lax.axis_index("c")
        ab = (a0, a1)
        bb_ = (b0, b1)
        ga = (ga0, ga1)
        gb = (gb0, gb1)
        ix = (ix0, ix1)

        def issue(t, p):
            chunk = t * NW + wid

            @pl.when((t < ROUNDS) & (chunk < NCHUNK))
            def _():
                base = chunk * CH

                @pl.when(t == 0)
                def _():
                    pltpu.sync_copy(src_hbm.at[pl.ds(base, CH)], ia.at[p])
                    pltpu.sync_copy(dst_hbm.at[pl.ds(base, CH)], ib.at[p])

                @pl.when(t > 0)
                def _():
                    pltpu.make_async_copy(src_hbm.at[pl.ds(base, CH)], ia.at[p], ix[p]).wait()
                    pltpu.make_async_copy(dst_hbm.at[pl.ds(base, CH)], ib.at[p], ix[p]).wait()

                pltpu.async_copy(ts_hbm.at[ia.at[p]], ab[p], ga[p])
                pltpu.async_copy(td_hbm.at[ib.at[p]], bb_[p], gb[p])

        def complete(t, p):
            chunk = t * NW + wid

            @pl.when((t >= 0) & (chunk < NCHUNK))
            def _():
                base = chunk * CH
                pltpu.make_async_copy(ts_hbm.at[pl.ds(0, CH)], ab[p], ga[p]).wait()
                pltpu.make_async_copy(td_hbm.at[pl.ds(0, CH)], bb_[p], gb[p]).wait()
                a_ref = ab[p]
                b_ref = bb_[p]

                def addrow(i, c2):
                    for j in range(8):
                        sl = pl.ds(j * 16, 16)
                        a_ref[i, sl] = a_ref[i, sl] + b_ref[i, sl]
                    return c2

                lax.fori_loop(0, CH, addrow, 0)
                pltpu.sync_copy(a_ref, out_hbm.at[pl.ds(base, CH)])

        def prefetch(t, p):
            chunk = t * NW + wid

            @pl.when((t < ROUNDS) & (chunk < NCHUNK))
            def _():
                base = chunk * CH
                pltpu.async_copy(src_hbm.at[pl.ds(base, CH)], ia.at[p], ix[p])
                pltpu.async_copy(dst_hbm.at[pl.ds(base, CH)], ib.at[p], ix[p])

        def round_(t, carry):
            for p in (0, 1):
                @pl.when((t % 2) == p)
                def _(p=p):
                    issue(t, p)
                    complete(t - 1, 1 - p)
                    prefetch(t + 1, 1 - p)
            return carry

        lax.fori_loop(0, ROUNDS + 1, round_, 0)

    return k(ts, td, src, dst)


def _sc_scatter(e32, dst):
    half = NCHUNK // 2

    @functools.partial(
        pl.kernel,
        out_type=jax.ShapeDtypeStruct((2 * SHN, 32), F32),
        mesh=_sc_mesh(),
        compiler_params=pltpu.CompilerParams(use_tc_tiling_on_sc=False),
        scratch_types=[
            pltpu.VMEM((2, CH), jnp.int32),
            pltpu.VMEM((CH, 32), F32),
            pltpu.VMEM((CH, 32), F32),
            pltpu.VMEM((NPS, 32), F32),
            pltpu.VMEM_SHARED((SHN, 32), F32),
            pltpu.SemaphoreType.DMA,
            pltpu.SemaphoreType.DMA,
            pltpu.SemaphoreType.DMA,
            pltpu.SemaphoreType.DMA,
        ],
    )
    def k(e_hbm, dst_hbm, out_hbm, idxb, r0, r1, obuf, shared,
          ix0, ix1, sa0, sa1):
        cid = lax.axis_index("c")
        sid = lax.axis_index("s")
        rows = (r0, r1)
        ix = (ix0, ix1)
        sa = (sa0, sa1)
        nrounds = (half + 15) // 16

        def zr(i, c):
            obuf[i, pl.ds(0, 16)] = jnp.zeros((16,), F32)
            obuf[i, pl.ds(16, 16)] = jnp.zeros((16,), F32)
            return c

        lax.fori_loop(0, NPS, zr, 0)
        pltpu.sync_copy(obuf, shared.at[pl.ds(sid * NPS, NPS)])
        plsc.subcore_barrier()

        def issue(t, p):
            cl = t * 16 + sid

            @pl.when((t < nrounds) & (cl < half))
            def _():
                base = (cid * half + cl) * CH

                @pl.when(t == 0)
                def _():
                    pltpu.sync_copy(dst_hbm.at[pl.ds(base, CH)], idxb.at[p])
                    pltpu.sync_copy(e_hbm.at[pl.ds(base, CH)], rows[p])

                @pl.when(t > 0)
                def _():
                    pltpu.make_async_copy(dst_hbm.at[pl.ds(base, CH)], idxb.at[p], ix[p]).wait()
                    pltpu.make_async_copy(e_hbm.at[pl.ds(base, CH)], rows[p], ix[p]).wait()

                pltpu.async_copy(rows[p], shared.at[idxb.at[p]], sa[p], add=True)

        def prefetch(t, p):
            cl = t * 16 + sid

            @pl.when((t < nrounds) & (cl < half))
            def _():
                base = (cid * half + cl) * CH
                @pl.when(t >= 2)
                def _():
                    pltpu.make_async_copy(e_hbm.at[pl.ds(0, CH)], rows[p], sa[p]).wait()

                pltpu.async_copy(dst_hbm.at[pl.ds(base, CH)], idxb.at[p], ix[p])
                pltpu.async_copy(e_hbm.at[pl.ds(base, CH)], rows[p], ix[p])

        def round_(t, carry):
            for p in (0, 1):
                @pl.when((t % 2) == p)
                def _(p=p):
                    issue(t, p)
                    prefetch(t + 1, 1 - p)
            return carry

        lax.fori_loop(0, nrounds, round_, 0)
        for p in (0, 1):
            pltpu.make_async_copy(e_hbm.at[pl.ds(0, CH)], rows[p], sa[p]).wait()
        plsc.subcore_barrier()
        pltpu.sync_copy(shared.at[pl.ds(sid * NPS, NPS)], obuf)
        pltpu.sync_copy(obuf, out_hbm.at[pl.ds(cid * SHN + sid * NPS, NPS)])

    return k(e32, dst)


def _node_body(u_ref, vu_ref, c1_ref, va_ref, vb_ref, v2_ref, c2_ref, v3_ref,
               c3_ref, x_ref, s0_ref, s1_ref, batch_ref, *rest, has_next):
    if has_next:
        wa_ref, wb_ref, xn_ref, xg_ref, cg_ref, tsp_ref, td_ref = rest
    else:
        xn_ref, xg_ref, cg_ref = rest
    s = s0_ref[...] + s1_ref[...]
    agg = s[:, :16] / jnp.maximum(s[:, 16:17], 1.0)
    oh = _onehot(batch_ref[0, 0, :])
    uw = _dot(u_ref[...], vu_ref[...]) + c1_ref[...]
    a1 = jnp.maximum(_dot(x_ref[...], va_ref[...]) + _dot(agg, vb_ref[...]) + _dot(oh, uw), 0.0)
    a2 = jnp.maximum(_dot(a1, v2_ref[...]) + c2_ref[...], 0.0)
    xn = _dot(a2, v3_ref[...]) + c3_ref[...]
    xn_ref[...] = xn
    pg = lax.dot_general(oh, xn, (((0,), (0,)), ((), ())), preferred_element_type=F32)
    cg = lax.dot_general(oh, jnp.ones_like(xn), (((0,), (0,)), ((), ())),
                         preferred_element_type=F32)

    @pl.when(pl.program_id(0) == 0)
    def _():
        xg_ref[...] = pg
        cg_ref[...] = cg

    @pl.when(pl.program_id(0) != 0)
    def _():
        xg_ref[...] = xg_ref[...] + pg
        cg_ref[...] = cg_ref[...] + cg

    if has_next:
        tsp_ref[...] = _dot(xn, wa_ref[...])
        td_ref[...] = _dot(xn, wb_ref[...])


def _node(x, sc, u, batch3, w, enext, has_next):
    cfull = lambda shape: pl.BlockSpec(shape, lambda i: tuple(0 for _ in shape))
    nblk = pl.BlockSpec((BN, 128), lambda i: (i, 0))
    accb = pl.BlockSpec((N_GRAPHS, 128), lambda i: (0, 0))
    in_specs = [
        cfull((N_GRAPHS, 128)), cfull((128, 128)), cfull((1, 128)),
        cfull((128, 128)), cfull((16, 128)), cfull((128, 128)), cfull((1, 128)),
        cfull((128, 128)), cfull((1, 128)),
        nblk,
        pl.BlockSpec((BN, 32), lambda i: (i, 0)),
        pl.BlockSpec((BN, 32), lambda i: (i, 0)),
        pl.BlockSpec((1, 1, BN), lambda i: (i, 0, 0)),
    ]
    args = [u, w['nvu'], w['nc1'], w['nva'], w['nvb'], w['nv2'], w['nc2'],
            w['nv3'], w['nc3'], x, sc[:N_NODES], sc[SHN:SHN + N_NODES], batch3]
    out_specs = [nblk, accb, accb]
    out_shape = [jax.ShapeDtypeStruct((N_NODES, 128), F32),
                 jax.ShapeDtypeStruct((N_GRAPHS, 128), F32),
                 jax.ShapeDtypeStruct((N_GRAPHS, 128), F32)]
    if has_next:
        in_specs += [cfull((128, 128)), cfull((128, 128))]
        args += [enext['ewa'], enext['ewb']]
        out_specs += [nblk, nblk]
        out_shape += [jax.ShapeDtypeStruct((N_NODES, 128), F32)] * 2
    return pl.pallas_call(
        functools.partial(_node_body, has_next=has_next),
        grid=(N_NODES // BN,),
        in_specs=in_specs, out_specs=out_specs, out_shape=out_shape,
    )(*args)


def _glob_body(xg1_ref, cg1_ref, u1_ref, xg2_ref, cg2_ref, u2_ref,
               ga1, gb1, h11, g21, h21, g31, h31,
               ga2, gb2, h12, g22, h22, g32, h32,
               ma, mb, n1, m2, n2, m3, n3, *rest, has_next):
    if has_next:
        wu1, eb1, wu2, eb2, u1n_ref, u2n_ref, out_ref, tsu1_ref, tsu2_ref = rest
    else:
        u1n_ref, u2n_ref, out_ref = rest

    def gmlp(xg_ref, cg_ref, u_ref, ga, gb, hb1, g2, hb2, g3, hb3):
        xg = xg_ref[...] / jnp.maximum(cg_ref[...], 1.0)
        t1 = jnp.maximum(_dot(xg, ga[...]) + _dot(u_ref[...], gb[...]) + hb1[...], 0.0)
        t2 = jnp.maximum(_dot(t1, g2[...]) + hb2[...], 0.0)
        return _dot(t2, g3[...]) + hb3[...]

    u1n = gmlp(xg1_ref, cg1_ref, u1_ref, ga1, gb1, h11, g21, h21, g31, h31)
    u2n = gmlp(xg2_ref, cg2_ref, u2_ref, ga2, gb2, h12, g22, h22, g32, h32)
    u1n_ref[...] = u1n
    u2n_ref[...] = u2n
    m1v = jnp.maximum(_dot(u1n, ma[...]) + _dot(u2n, mb[...]) + n1[...], 0.0)
    m2v = jnp.maximum(_dot(m1v, m2[...]) + n2[...], 0.0)
    out_ref[...] = _dot(m2v, m3[...]) + n3[...]
    if has_next:
        tsu1_ref[...] = _dot(u1n, wu1[...]) + eb1[...]
        tsu2_ref[...] = _dot(u2n, wu2[...]) + eb2[...]


def _glob(xg1, cg1, u1, xg2, cg2, u2, w1, w2, mw, e1w, e2w, has_next):
    args = [xg1, cg1, u1, xg2, cg2, u2,
            w1['gga'], w1['ggb'], w1['gd1'], w1['gg2'], w1['gd2'], w1['gg3'], w1['gd3'],
            w2['gga'], w2['ggb'], w2['gd1'], w2['gg2'], w2['gd2'], w2['gg3'], w2['gd3'],
            mw['ma'], mw['mb'], mw['n1'], mw['m2'], mw['n2'], mw['m3p'], mw['n3p']]
    nout = 3
    out_shape = [jax.ShapeDtypeStruct((N_GRAPHS, 128), F32)] * 3
    if has_next:
        args += [e1w['ewu'], e1w['eb1'], e2w['ewu'], e2w['eb1']]
        out_shape += [jax.ShapeDtypeStruct((N_GRAPHS, 128), F32)] * 2
        nout = 5
    return pl.pallas_call(
        functools.partial(_glob_body, has_next=has_next),
        out_shape=out_shape,
    )(*args)


def _tfix_body(tsu1_ref, tsu2_ref, tsp1_ref, b1_ref, tsp2_ref, b2_ref,
               ts1_ref, ts2_ref):
    oh1 = _onehot(b1_ref[0, 0, :])
    oh2 = _onehot(b2_ref[0, 0, :])
    ts1_ref[...] = tsp1_ref[...] + _dot(oh1, tsu1_ref[...])
    ts2_ref[...] = tsp2_ref[...] + _dot(oh2, tsu2_ref[...])


def _tfix(tsp1, batch31, tsu1, tsp2, batch32, tsu2):
    cfull = lambda shape: pl.BlockSpec(shape, lambda i: tuple(0 for _ in shape))
    nblk = pl.BlockSpec((BN, 128), lambda i: (i, 0))
    bblk = pl.BlockSpec((1, 1, BN), lambda i: (i, 0, 0))
    return pl.pallas_call(
        _tfix_body,
        grid=(N_NODES // BN,),
        in_specs=[cfull((N_GRAPHS, 128)), cfull((N_GRAPHS, 128)),
                  nblk, bblk, nblk, bblk],
        out_specs=[nblk, nblk],
        out_shape=[jax.ShapeDtypeStruct((N_NODES, 128), F32)] * 2,
    )(tsu1, tsu2, tsp1, batch31, tsp2, batch32)


def _unpack_block(p):
    (W1, b1), (W2, b2), (W3, b3) = p['edge']
    (V1, c1), (V2, c2), (V3, c3) = p['node']
    (G1, d1), (G2, d2), (G3, d3) = p['global']
    return dict(
        ewa=W1[0:128], ewb=W1[128:256], ewc=W1[256:272], ewu=W1[272:400],
        eb1=b1.reshape(1, 128), ew2=W2, eb2=b2.reshape(1, 128),
        ew3=W3, eb3=b3.reshape(1, 16),
        nva=V1[0:128], nvb=V1[128:144], nvu=V1[144:272], nc1=c1.reshape(1, 128),
        nv2=V2, nc2=c2.reshape(1, 128), nv3=V3, nc3=c3.reshape(1, 128),
        gga=G1[0:128], ggb=G1[128:256], gd1=d1.reshape(1, 128),
        gg2=G2, gd2=d2.reshape(1, 128), gg3=G3, gd3=d3.reshape(1, 128),
    )


def _unpack_mlp(pm):
    (M1, n1), (M2, n2), (M3, n3) = pm
    m3p = jnp.zeros((128, 128), F32).at[:, :2].set(M3)
    n3p = jnp.zeros((1, 128), F32).at[0, :2].set(n3)
    return dict(ma=M1[:128], mb=M1[128:], n1=n1.reshape(1, 128),
                m2=M2, n2=n2.reshape(1, 128), m3p=m3p, n3p=n3p)


def kernel(x1, edge_index1, e1, u1, batch1, x2, edge_index2, e2, u2, batch2, params):
    w1 = _unpack_block(params['gnn1'])
    w2 = _unpack_block(params['gnn2'])
    mw = _unpack_mlp(params['mlp'])
    s1, d1 = edge_index1[0], edge_index1[1]
    s2, d2 = edge_index2[0], edge_index2[1]
    b31 = batch1.reshape(N_NODES // BN, 1, BN)
    b32 = batch2.reshape(N_NODES // BN, 1, BN)

    ts1, td1 = _prep(x1, b31, u1, w1['ewa'], w1['ewb'], w1['ewu'], w1['eb1'])
    ts2, td2 = _prep(x2, b32, u2, w2['ewa'], w2['ewb'], w2['ewu'], w2['eb1'])

    outs = []
    e1c, e2c = e1, e2
    x1c, x2c = x1, x2
    u1c, u2c = u1, u2
    for step in range(2):
        has_next = step == 0
        g1 = _sc_gather(ts1, td1, s1, d1)
        g2 = _sc_gather(ts2, td2, s2, d2)
        en1 = _edge(g1, e1c, w1['ewc'], w1['ew2'], w1['eb2'], w1['ew3'], w1['eb3'])
        en2 = _edge(g2, e2c, w2['ewc'], w2['ew2'], w2['eb2'], w2['ew3'], w2['eb3'])
        sc1 = _sc_scatter(en1, d1)
        sc2 = _sc_scatter(en2, d2)
        r1 = _node(x1c, sc1, u1c, b31, w1, w1, has_next)
        r2 = _node(x2c, sc2, u2c, b32, w2, w2, has_next)
        gout = _glob(r1[1], r1[2], u1c, r2[1], r2[2], u2c, w1, w2, mw, w1, w2, has_next)
        outs.append(gout[2][:, :2])
        if has_next:
            ts1, ts2 = _tfix(r1[3], b31, gout[3], r2[3], b32, gout[4])
            td1, td2 = r1[4], r2[4]
        x1c, x2c = r1[0], r2[0]
        u1c, u2c = gout[0], gout[1]
        e1c, e2c = en1, en2
    return jnp.stack(outs)
